# baseline (speedup 1.0000x reference)
import functools

import jax
import jax.numpy as jnp
from jax import lax
from jax.experimental import pallas as pl
from jax.experimental.pallas import tpu as pltpu
from jax.experimental.pallas import tpu_sc as plsc

NC = 2
NS = 16
NW = NC * NS

ROWS_MAIN = 632
ROWS_LAST = 10000 - 15 * ROWS_MAIN

CH = 128


def _row_partition_copy(src_ref, dst_ref, sid, src_at_zero=False):
    @pl.when(sid < NS - 1)
    def _():
        s0 = 0 if src_at_zero else sid * ROWS_MAIN
        pltpu.sync_copy(
            src_ref.at[pl.ds(s0, ROWS_MAIN)],
            dst_ref.at[pl.ds(sid * ROWS_MAIN, ROWS_MAIN)],
        )

    @pl.when(sid == NS - 1)
    def _():
        s0 = 0 if src_at_zero else (NS - 1) * ROWS_MAIN
        pltpu.sync_copy(
            src_ref.at[pl.ds(s0, ROWS_LAST)],
            dst_ref.at[pl.ds((NS - 1) * ROWS_MAIN, ROWS_LAST)],
        )


def _padded_chunks(e):
    quantum = NW * CH * 4
    e_pad = -(-e // quantum) * quantum
    return e_pad, (e_pad // CH) // NW


def _make_sc_spmm(n, e, d):
    e_pad, per_w = _padded_chunks(e)
    half = per_w // 2

    mesh = plsc.VectorSubcoreMesh(core_axis_name="c", subcore_axis_name="s")

    @functools.partial(
        pl.kernel,
        mesh=mesh,
        out_type=jax.ShapeDtypeStruct((NC, n, d), jnp.float32),
        scratch_types=[
            pltpu.VMEM((half, CH), jnp.int32),
            pltpu.VMEM((half, CH), jnp.int32),
            [pltpu.VMEM((CH, d), jnp.float32) for _ in range(2)],
            pltpu.VMEM_SHARED((n + 8, d), jnp.float32),
            [pltpu.SemaphoreType.DMA for _ in range(2)],
            [pltpu.SemaphoreType.DMA for _ in range(2)],
        ],
    )
    def spmm(src_hbm, dst_hbm, g_hbm, zeros_hbm, out_hbm,
             isv, idv, rows, acc, gsem, ssem):
        cid = lax.axis_index("c")
        sid = lax.axis_index("s")
        wid = cid * NS + sid

        _row_partition_copy(zeros_hbm, acc, sid, src_at_zero=True)
        plsc.subcore_barrier()

        def start_gather(j, b):
            pltpu.async_copy(g_hbm.at[isv.at[j]], rows[b], gsem[b])

        def wait_gather(j, b):
            pltpu.make_async_copy(g_hbm.at[isv.at[j]], rows[b], gsem[b]).wait()

        def start_scatter(j, b):
            pltpu.async_copy(rows[b], acc.at[idv.at[j]], ssem[b], add=True)

        def wait_scatter(j, b):
            pltpu.make_async_copy(rows[b], acc.at[idv.at[j]], ssem[b]).wait()

        for h in range(2):
            pltpu.sync_copy(
                src_hbm.at[pl.ds((wid * 2 + h) * half, half)], isv)
            pltpu.sync_copy(
                dst_hbm.at[pl.ds((wid * 2 + h) * half, half)], idv)

            start_gather(0, 0)
            start_gather(1, 1)
            wait_gather(0, 0)
            start_scatter(0, 0)

            def step(k, carry):
                for t in range(2):
                    j = 2 * k + 1 + t
                    b = (1 + t) % 2
                    wait_scatter(j - 1, 1 - b)
                    start_gather(j + 1, 1 - b)
                    wait_gather(j, b)
                    start_scatter(j, b)
                return carry

            lax.fori_loop(0, (half - 2) // 2, step, 0)

            j0 = half - 1
            wait_scatter(j0 - 1, 0)
            wait_gather(j0, 1)
            start_scatter(j0, 1)
            wait_scatter(j0, 1)

        plsc.subcore_barrier()
        _row_partition_copy(acc, out_hbm.at[cid], sid)

    return spmm


def _make_sc_degree(n, e):
    _, per_w = _padded_chunks(e)

    mesh = plsc.VectorSubcoreMesh(core_axis_name="c", subcore_axis_name="s")

    @functools.partial(
        pl.kernel,
        mesh=mesh,
        out_type=jax.ShapeDtypeStruct((NC, n, 128), jnp.float32),
        scratch_types=[
            pltpu.VMEM((per_w, CH), jnp.int32),
            pltpu.VMEM((CH, 128), jnp.float32),
            pltpu.VMEM_SHARED((n + 8, 128), jnp.float32),
            [pltpu.SemaphoreType.DMA for _ in range(2)],
        ],
    )
    def degree(dst_hbm, ones_hbm, zeros_hbm, out_hbm,
               idv, ones_v, acc, ssem):
        cid = lax.axis_index("c")
        sid = lax.axis_index("s")
        wid = cid * NS + sid

        _row_partition_copy(zeros_hbm, acc, sid, src_at_zero=True)
        pltpu.sync_copy(ones_hbm, ones_v)
        pltpu.sync_copy(dst_hbm.at[pl.ds(wid * per_w, per_w)], idv)
        plsc.subcore_barrier()

        def start_scatter(j, b):
            pltpu.async_copy(ones_v, acc.at[idv.at[j]], ssem[b], add=True)

        def wait_scatter(j, b):
            pltpu.make_async_copy(ones_v, acc.at[idv.at[j]], ssem[b]).wait()

        start_scatter(0, 0)
        start_scatter(1, 1)

        def step(k, carry):
            for t in range(2):
                j = 2 * k + 2 + t
                wait_scatter(j - 2, t)
                start_scatter(j, t)
            return carry

        lax.fori_loop(0, (per_w - 2) // 2, step, 0)
        wait_scatter(per_w - 2, 0)
        wait_scatter(per_w - 1, 1)

        plsc.subcore_barrier()
        _row_partition_copy(acc, out_hbm.at[cid], sid)

    return degree


BLK = 1000


def _tc_mm_body(x_ref, w_ref, out_ref):
    out_ref[...] = jnp.dot(
        x_ref[...], w_ref[...], preferred_element_type=jnp.float32)


def _tc_scale_body(dp_ref, h_ref, g_ref, dinv_ref):
    deg = dp_ref[0, :, :1] + dp_ref[1, :, :1] + 1.0
    dinv = lax.rsqrt(deg)
    dinv_ref[...] = dinv
    g_ref[...] = dinv * h_ref[...]


def _tc_mid_body(p_ref, g_ref, dinv_ref, b_ref, w_ref, out_ref):
    dinv = dinv_ref[...]
    h = p_ref[0] + p_ref[1] + g_ref[...]
    h = jnp.maximum(dinv * h + b_ref[...], 0.0)
    out_ref[...] = dinv * jnp.dot(
        h, w_ref[...], preferred_element_type=jnp.float32)


def _tc_last_body(d_out, p_ref, g_ref, dinv_ref, b_ref, out_ref):
    z = p_ref[0, :, :d_out] + p_ref[1, :, :d_out] + g_ref[:, :d_out]
    z = jnp.maximum(dinv_ref[...] * z + b_ref[...], 0.0)
    m = jnp.max(z, axis=1, keepdims=True)
    lse = m + jnp.log(jnp.sum(jnp.exp(z - m), axis=1, keepdims=True))
    out_ref[...] = z - lse


def _tc_mm(x, w):
    n, d_in = x.shape
    d_out = w.shape[1]
    return pl.pallas_call(
        _tc_mm_body,
        grid=(n // BLK,),
        in_specs=[
            pl.BlockSpec((BLK, d_in), lambda i: (i, 0)),
            pl.BlockSpec((d_in, d_out), lambda i: (0, 0)),
        ],
        out_specs=pl.BlockSpec((BLK, d_out), lambda i: (i, 0)),
        out_shape=jax.ShapeDtypeStruct((n, d_out), jnp.float32),
    )(x, w)


def _tc_scale(dp, h):
    n, d = h.shape
    return pl.pallas_call(
        _tc_scale_body,
        grid=(n // BLK,),
        in_specs=[
            pl.BlockSpec((NC, BLK, dp.shape[2]), lambda i: (0, i, 0)),
            pl.BlockSpec((BLK, d), lambda i: (i, 0)),
        ],
        out_specs=[
            pl.BlockSpec((BLK, d), lambda i: (i, 0)),
            pl.BlockSpec((BLK, 1), lambda i: (i, 0)),
        ],
        out_shape=[
            jax.ShapeDtypeStruct((n, d), jnp.float32),
            jax.ShapeDtypeStruct((n, 1), jnp.float32),
        ],
    )(dp, h)


def _tc_mid(p, g, dinv, b, w):
    n, d = g.shape
    d_out = w.shape[1]
    return pl.pallas_call(
        _tc_mid_body,
        grid=(n // BLK,),
        in_specs=[
            pl.BlockSpec((NC, BLK, d), lambda i: (0, i, 0)),
            pl.BlockSpec((BLK, d), lambda i: (i, 0)),
            pl.BlockSpec((BLK, 1), lambda i: (i, 0)),
            pl.BlockSpec((1, d), lambda i: (0, 0)),
            pl.BlockSpec((d, d_out), lambda i: (0, 0)),
        ],
        out_specs=pl.BlockSpec((BLK, d_out), lambda i: (i, 0)),
        out_shape=jax.ShapeDtypeStruct((n, d_out), jnp.float32),
    )(p, g, dinv, b, w)


def _tc_last(p, g, dinv, b, d_out):
    n, d = g.shape
    return pl.pallas_call(
        functools.partial(_tc_last_body, d_out),
        grid=(n // BLK,),
        in_specs=[
            pl.BlockSpec((NC, BLK, d), lambda i: (0, i, 0)),
            pl.BlockSpec((BLK, d), lambda i: (i, 0)),
            pl.BlockSpec((BLK, 1), lambda i: (i, 0)),
            pl.BlockSpec((1, d_out), lambda i: (0, 0)),
        ],
        out_specs=pl.BlockSpec((BLK, d_out), lambda i: (i, 0)),
        out_shape=jax.ShapeDtypeStruct((n, d_out), jnp.float32),
    )(p, g, dinv, b)


def kernel(x, edge_index, W1, b1, W2, b2, W3, b3):
    n, d_in = x.shape
    e = edge_index.shape[1]
    d_hid = W2.shape[0]
    d_out = W3.shape[1]

    e_pad, _ = _padded_chunks(e)
    src2 = jnp.concatenate(
        [edge_index[0], jnp.zeros((e_pad - e,), jnp.int32)]).reshape(-1, CH)
    dst2 = jnp.concatenate(
        [edge_index[1], jnp.full((e_pad - e,), n, jnp.int32)]).reshape(-1, CH)

    zeros_wide = jnp.zeros((ROWS_MAIN, max(d_in, d_hid)), jnp.float32)
    ones128 = jnp.ones((CH, 128), jnp.float32)

    sc_degree = _make_sc_degree(n, e)
    sc_spmm_h = _make_sc_spmm(n, e, d_hid)

    W3p = jnp.pad(W3, ((0, 0), (0, d_hid - d_out)))

    dp = sc_degree(dst2, ones128, zeros_wide)
    h1 = _tc_mm(x, W1)
    g1, dinv = _tc_scale(dp, h1)
    p1 = sc_spmm_h(src2, dst2, g1, zeros_wide)
    g2 = _tc_mid(p1, g1, dinv, b1.reshape(1, -1), W2)
    p2 = sc_spmm_h(src2, dst2, g2, zeros_wide)
    g3 = _tc_mid(p2, g2, dinv, b2.reshape(1, -1), W3p)
    p3 = sc_spmm_h(src2, dst2, g3, zeros_wide)
    return _tc_last(p3, g3, dinv, b3.reshape(1, -1), d_out)


# speedup vs baseline: 1.0003x; 1.0003x over previous
import functools

import jax
import jax.numpy as jnp
from jax import lax
from jax.experimental import pallas as pl
from jax.experimental.pallas import tpu as pltpu
from jax.experimental.pallas import tpu_sc as plsc

NC = 2
NS = 16
NW = NC * NS

ROWS_MAIN = 632
ROWS_LAST = 10000 - 15 * ROWS_MAIN

CH = 128


def _row_partition_copy(src_ref, dst_ref, sid, src_at_zero=False):
    @pl.when(sid < NS - 1)
    def _():
        s0 = 0 if src_at_zero else sid * ROWS_MAIN
        pltpu.sync_copy(
            src_ref.at[pl.ds(s0, ROWS_MAIN)],
            dst_ref.at[pl.ds(sid * ROWS_MAIN, ROWS_MAIN)],
        )

    @pl.when(sid == NS - 1)
    def _():
        s0 = 0 if src_at_zero else (NS - 1) * ROWS_MAIN
        pltpu.sync_copy(
            src_ref.at[pl.ds(s0, ROWS_LAST)],
            dst_ref.at[pl.ds((NS - 1) * ROWS_MAIN, ROWS_LAST)],
        )


def _padded_chunks(e):
    quantum = NW * CH * 4
    e_pad = -(-e // quantum) * quantum
    return e_pad, (e_pad // CH) // NW


def _make_sc_spmm(n, e, d):
    e_pad, per_w = _padded_chunks(e)
    half = per_w // 2

    mesh = plsc.VectorSubcoreMesh(core_axis_name="c", subcore_axis_name="s")

    @functools.partial(
        pl.kernel,
        mesh=mesh,
        out_type=jax.ShapeDtypeStruct((NC, n, d), jnp.float32),
        scratch_types=[
            pltpu.VMEM((half, CH), jnp.int32),
            pltpu.VMEM((half, CH), jnp.int32),
            [pltpu.VMEM((CH, d), jnp.float32) for _ in range(2)],
            pltpu.VMEM_SHARED((n + 128, d), jnp.float32),
            [pltpu.SemaphoreType.DMA for _ in range(2)],
            [pltpu.SemaphoreType.DMA for _ in range(2)],
        ],
    )
    def spmm(src_hbm, dst_hbm, g_hbm, zeros_hbm, out_hbm,
             isv, idv, rows, acc, gsem, ssem):
        cid = lax.axis_index("c")
        sid = lax.axis_index("s")
        wid = cid * NS + sid

        _row_partition_copy(zeros_hbm, acc, sid, src_at_zero=True)
        plsc.subcore_barrier()

        def start_gather(j, b):
            pltpu.async_copy(g_hbm.at[isv.at[j]], rows[b], gsem[b])

        def wait_gather(j, b):
            pltpu.make_async_copy(g_hbm.at[isv.at[j]], rows[b], gsem[b]).wait()

        def start_scatter(j, b):
            pltpu.async_copy(rows[b], acc.at[idv.at[j]], ssem[b], add=True)

        def wait_scatter(j, b):
            pltpu.make_async_copy(rows[b], acc.at[idv.at[j]], ssem[b]).wait()

        for h in range(2):
            pltpu.sync_copy(
                src_hbm.at[pl.ds((wid * 2 + h) * half, half)], isv)
            pltpu.sync_copy(
                dst_hbm.at[pl.ds((wid * 2 + h) * half, half)], idv)

            start_gather(0, 0)
            start_gather(1, 1)
            wait_gather(0, 0)
            start_scatter(0, 0)

            def step(k, carry):
                for t in range(2):
                    j = 2 * k + 1 + t
                    b = (1 + t) % 2
                    wait_scatter(j - 1, 1 - b)
                    start_gather(j + 1, 1 - b)
                    wait_gather(j, b)
                    start_scatter(j, b)
                return carry

            lax.fori_loop(0, (half - 2) // 2, step, 0)

            j0 = half - 1
            wait_scatter(j0 - 1, 0)
            wait_gather(j0, 1)
            start_scatter(j0, 1)
            wait_scatter(j0, 1)

        plsc.subcore_barrier()
        _row_partition_copy(acc, out_hbm.at[cid], sid)

    return spmm


def _make_sc_degree(n, e):
    _, per_w = _padded_chunks(e)

    mesh = plsc.VectorSubcoreMesh(core_axis_name="c", subcore_axis_name="s")

    @functools.partial(
        pl.kernel,
        mesh=mesh,
        out_type=jax.ShapeDtypeStruct((NC, n, 128), jnp.float32),
        scratch_types=[
            pltpu.VMEM((per_w, CH), jnp.int32),
            pltpu.VMEM((CH, 128), jnp.float32),
            pltpu.VMEM_SHARED((n + 128, 128), jnp.float32),
            [pltpu.SemaphoreType.DMA for _ in range(2)],
        ],
    )
    def degree(dst_hbm, ones_hbm, zeros_hbm, out_hbm,
               idv, ones_v, acc, ssem):
        cid = lax.axis_index("c")
        sid = lax.axis_index("s")
        wid = cid * NS + sid

        _row_partition_copy(zeros_hbm, acc, sid, src_at_zero=True)
        pltpu.sync_copy(ones_hbm, ones_v)
        pltpu.sync_copy(dst_hbm.at[pl.ds(wid * per_w, per_w)], idv)
        plsc.subcore_barrier()

        def start_scatter(j, b):
            pltpu.async_copy(ones_v, acc.at[idv.at[j]], ssem[b], add=True)

        def wait_scatter(j, b):
            pltpu.make_async_copy(ones_v, acc.at[idv.at[j]], ssem[b]).wait()

        start_scatter(0, 0)
        start_scatter(1, 1)

        def step(k, carry):
            for t in range(2):
                j = 2 * k + 2 + t
                wait_scatter(j - 2, t)
                start_scatter(j, t)
            return carry

        lax.fori_loop(0, (per_w - 2) // 2, step, 0)
        wait_scatter(per_w - 2, 0)
        wait_scatter(per_w - 1, 1)

        plsc.subcore_barrier()
        _row_partition_copy(acc, out_hbm.at[cid], sid)

    return degree


BLK = 1000


def _tc_mm_body(x_ref, w_ref, out_ref):
    out_ref[...] = jnp.dot(
        x_ref[...], w_ref[...], preferred_element_type=jnp.float32)


def _tc_scale_body(dp_ref, h_ref, g_ref, dinv_ref):
    deg = dp_ref[0, :, :1] + dp_ref[1, :, :1] + 1.0
    dinv = lax.rsqrt(deg)
    dinv_ref[...] = dinv
    g_ref[...] = dinv * h_ref[...]


def _tc_mid_body(p_ref, g_ref, dinv_ref, b_ref, w_ref, out_ref):
    dinv = dinv_ref[...]
    h = p_ref[0] + p_ref[1] + g_ref[...]
    h = jnp.maximum(dinv * h + b_ref[...], 0.0)
    out_ref[...] = dinv * jnp.dot(
        h, w_ref[...], preferred_element_type=jnp.float32)


def _tc_last_body(d_out, p_ref, g_ref, dinv_ref, b_ref, out_ref):
    z = p_ref[0, :, :d_out] + p_ref[1, :, :d_out] + g_ref[:, :d_out]
    z = jnp.maximum(dinv_ref[...] * z + b_ref[...], 0.0)
    m = jnp.max(z, axis=1, keepdims=True)
    lse = m + jnp.log(jnp.sum(jnp.exp(z - m), axis=1, keepdims=True))
    out_ref[...] = z - lse


def _tc_mm(x, w):
    n, d_in = x.shape
    d_out = w.shape[1]
    return pl.pallas_call(
        _tc_mm_body,
        grid=(n // BLK,),
        in_specs=[
            pl.BlockSpec((BLK, d_in), lambda i: (i, 0)),
            pl.BlockSpec((d_in, d_out), lambda i: (0, 0)),
        ],
        out_specs=pl.BlockSpec((BLK, d_out), lambda i: (i, 0)),
        out_shape=jax.ShapeDtypeStruct((n, d_out), jnp.float32),
    )(x, w)


def _tc_scale(dp, h):
    n, d = h.shape
    return pl.pallas_call(
        _tc_scale_body,
        grid=(n // BLK,),
        in_specs=[
            pl.BlockSpec((NC, BLK, dp.shape[2]), lambda i: (0, i, 0)),
            pl.BlockSpec((BLK, d), lambda i: (i, 0)),
        ],
        out_specs=[
            pl.BlockSpec((BLK, d), lambda i: (i, 0)),
            pl.BlockSpec((BLK, 1), lambda i: (i, 0)),
        ],
        out_shape=[
            jax.ShapeDtypeStruct((n, d), jnp.float32),
            jax.ShapeDtypeStruct((n, 1), jnp.float32),
        ],
    )(dp, h)


def _tc_mid(p, g, dinv, b, w):
    n, d = g.shape
    d_out = w.shape[1]
    return pl.pallas_call(
        _tc_mid_body,
        grid=(n // BLK,),
        in_specs=[
            pl.BlockSpec((NC, BLK, d), lambda i: (0, i, 0)),
            pl.BlockSpec((BLK, d), lambda i: (i, 0)),
            pl.BlockSpec((BLK, 1), lambda i: (i, 0)),
            pl.BlockSpec((1, d), lambda i: (0, 0)),
            pl.BlockSpec((d, d_out), lambda i: (0, 0)),
        ],
        out_specs=pl.BlockSpec((BLK, d_out), lambda i: (i, 0)),
        out_shape=jax.ShapeDtypeStruct((n, d_out), jnp.float32),
    )(p, g, dinv, b, w)


def _tc_last(p, g, dinv, b, d_out):
    n, d = g.shape
    return pl.pallas_call(
        functools.partial(_tc_last_body, d_out),
        grid=(n // BLK,),
        in_specs=[
            pl.BlockSpec((NC, BLK, d), lambda i: (0, i, 0)),
            pl.BlockSpec((BLK, d), lambda i: (i, 0)),
            pl.BlockSpec((BLK, 1), lambda i: (i, 0)),
            pl.BlockSpec((1, d_out), lambda i: (0, 0)),
        ],
        out_specs=pl.BlockSpec((BLK, d_out), lambda i: (i, 0)),
        out_shape=jax.ShapeDtypeStruct((n, d_out), jnp.float32),
    )(p, g, dinv, b)


def kernel(x, edge_index, W1, b1, W2, b2, W3, b3):
    n, d_in = x.shape
    e = edge_index.shape[1]
    d_hid = W2.shape[0]
    d_out = W3.shape[1]

    e_pad, _ = _padded_chunks(e)
    src2 = jnp.concatenate(
        [edge_index[0], jnp.zeros((e_pad - e,), jnp.int32)]).reshape(-1, CH)
    pad_dst = n + (jnp.arange(e_pad - e, dtype=jnp.int32) % 128)
    dst2 = jnp.concatenate([edge_index[1], pad_dst]).reshape(-1, CH)

    zeros_wide = jnp.zeros((ROWS_MAIN, max(d_in, d_hid)), jnp.float32)
    ones128 = jnp.ones((CH, 128), jnp.float32)

    sc_degree = _make_sc_degree(n, e)
    sc_spmm_h = _make_sc_spmm(n, e, d_hid)

    W3p = jnp.pad(W3, ((0, 0), (0, d_hid - d_out)))

    dp = sc_degree(dst2, ones128, zeros_wide)
    h1 = _tc_mm(x, W1)
    g1, dinv = _tc_scale(dp, h1)
    p1 = sc_spmm_h(src2, dst2, g1, zeros_wide)
    g2 = _tc_mid(p1, g1, dinv, b1.reshape(1, -1), W2)
    p2 = sc_spmm_h(src2, dst2, g2, zeros_wide)
    g3 = _tc_mid(p2, g2, dinv, b2.reshape(1, -1), W3p)
    p3 = sc_spmm_h(src2, dst2, g3, zeros_wide)
    return _tc_last(p3, g3, dinv, b3.reshape(1, -1), d_out)


# speedup vs baseline: 3.1054x; 3.1043x over previous
import functools

import jax
import jax.numpy as jnp
from jax import lax
from jax.experimental import pallas as pl
from jax.experimental.pallas import tpu as pltpu
from jax.experimental.pallas import tpu_sc as plsc

NC = 2
NS = 16
NW = NC * NS

ROWS_MAIN = 632
ROWS_LAST = 10000 - 15 * ROWS_MAIN

CH = 128


def _row_partition_copy(src_ref, dst_ref, sid, src_at_zero=False):
    @pl.when(sid < NS - 1)
    def _():
        s0 = 0 if src_at_zero else sid * ROWS_MAIN
        pltpu.sync_copy(
            src_ref.at[pl.ds(s0, ROWS_MAIN)],
            dst_ref.at[pl.ds(sid * ROWS_MAIN, ROWS_MAIN)],
        )

    @pl.when(sid == NS - 1)
    def _():
        s0 = 0 if src_at_zero else (NS - 1) * ROWS_MAIN
        pltpu.sync_copy(
            src_ref.at[pl.ds(s0, ROWS_LAST)],
            dst_ref.at[pl.ds((NS - 1) * ROWS_MAIN, ROWS_LAST)],
        )


def _padded_chunks(e):
    quantum = NW * CH * 4
    e_pad = -(-e // quantum) * quantum
    return e_pad, (e_pad // CH) // NW


def _make_sc_spmm(n, e, d):
    e_pad, per_w = _padded_chunks(e)
    half = per_w // 2

    mesh = plsc.VectorSubcoreMesh(core_axis_name="c", subcore_axis_name="s")

    @functools.partial(
        pl.kernel,
        mesh=mesh,
        out_type=jax.ShapeDtypeStruct((NC, n, d), jnp.float32),
        scratch_types=[
            pltpu.VMEM((half, CH), jnp.int32),
            pltpu.VMEM((half, CH), jnp.int32),
            [pltpu.VMEM((CH, d), jnp.float32) for _ in range(2)],
            pltpu.VMEM_SHARED((n + 128, d), jnp.float32),
            [pltpu.SemaphoreType.DMA for _ in range(2)],
            [pltpu.SemaphoreType.DMA for _ in range(2)],
        ],
    )
    def spmm(src_hbm, dst_hbm, g_hbm, zeros_hbm, out_hbm,
             isv, idv, rows, acc, gsem, ssem):
        cid = lax.axis_index("c")
        sid = lax.axis_index("s")
        wid = cid * NS + sid

        _row_partition_copy(zeros_hbm, acc, sid, src_at_zero=True)
        plsc.subcore_barrier()

        def start_gather(j, b):
            pltpu.async_copy(g_hbm.at[isv.at[j]], rows[b], gsem[b])

        def wait_gather(j, b):
            pltpu.make_async_copy(g_hbm.at[isv.at[j]], rows[b], gsem[b]).wait()

        def start_scatter(j, b):
            pltpu.async_copy(rows[b], acc.at[idv.at[j]], ssem[b], add=True)

        def wait_scatter(j, b):
            pltpu.make_async_copy(rows[b], acc.at[idv.at[j]], ssem[b]).wait()

        for h in range(2):
            pltpu.sync_copy(
                src_hbm.at[pl.ds((wid * 2 + h) * half, half)], isv)
            pltpu.sync_copy(
                dst_hbm.at[pl.ds((wid * 2 + h) * half, half)], idv)

            start_gather(0, 0)
            start_gather(1, 1)
            wait_gather(0, 0)
            start_scatter(0, 0)

            def step(k, carry):
                for t in range(2):
                    j = 2 * k + 1 + t
                    b = (1 + t) % 2
                    wait_scatter(j - 1, 1 - b)
                    start_gather(j + 1, 1 - b)
                    wait_gather(j, b)
                    start_scatter(j, b)
                return carry

            lax.fori_loop(0, (half - 2) // 2, step, 0)

            j0 = half - 1
            wait_scatter(j0 - 1, 0)
            wait_gather(j0, 1)
            start_scatter(j0, 1)
            wait_scatter(j0, 1)

        plsc.subcore_barrier()
        _row_partition_copy(acc, out_hbm.at[cid], sid)

    return spmm


def _make_sc_degree(n, e):
    _, per_w = _padded_chunks(e)

    mesh = plsc.VectorSubcoreMesh(core_axis_name="c", subcore_axis_name="s")

    @functools.partial(
        pl.kernel,
        mesh=mesh,
        out_type=jax.ShapeDtypeStruct((NC, n, 128), jnp.float32),
        scratch_types=[
            pltpu.VMEM((per_w, CH), jnp.int32),
            pltpu.VMEM((CH, 128), jnp.float32),
            pltpu.VMEM_SHARED((n + 128, 128), jnp.float32),
            [pltpu.SemaphoreType.DMA for _ in range(2)],
        ],
    )
    def degree(dst_hbm, ones_hbm, zeros_hbm, out_hbm,
               idv, ones_v, acc, ssem):
        cid = lax.axis_index("c")
        sid = lax.axis_index("s")
        wid = cid * NS + sid

        _row_partition_copy(zeros_hbm, acc, sid, src_at_zero=True)
        pltpu.sync_copy(ones_hbm, ones_v)
        pltpu.sync_copy(dst_hbm.at[pl.ds(wid * per_w, per_w)], idv)
        plsc.subcore_barrier()

        def start_scatter(j, b):
            pltpu.async_copy(ones_v, acc.at[idv.at[j]], ssem[b], add=True)

        def wait_scatter(j, b):
            pltpu.make_async_copy(ones_v, acc.at[idv.at[j]], ssem[b]).wait()

        start_scatter(0, 0)
        start_scatter(1, 1)

        def step(k, carry):
            for t in range(2):
                j = 2 * k + 2 + t
                wait_scatter(j - 2, t)
                start_scatter(j, t)
            return carry

        lax.fori_loop(0, (per_w - 2) // 2, step, 0)
        wait_scatter(per_w - 2, 0)
        wait_scatter(per_w - 1, 1)

        plsc.subcore_barrier()
        _row_partition_copy(acc, out_hbm.at[cid], sid)

    return degree


BLK = 1000


def _tc_mm_body(x_ref, w_ref, out_ref):
    out_ref[...] = jnp.dot(
        x_ref[...], w_ref[...], preferred_element_type=jnp.float32)


def _tc_scale_body(dp_ref, h_ref, g_ref, dinv_ref):
    deg = dp_ref[0, :, :1] + dp_ref[1, :, :1] + 1.0
    dinv = lax.rsqrt(deg)
    dinv_ref[...] = dinv
    g_ref[...] = dinv * h_ref[...]


def _tc_mid_body(p_ref, g_ref, dinv_ref, b_ref, w_ref, out_ref):
    dinv = dinv_ref[...]
    h = p_ref[0] + p_ref[1] + g_ref[...]
    h = jnp.maximum(dinv * h + b_ref[...], 0.0)
    out_ref[...] = dinv * jnp.dot(
        h, w_ref[...], preferred_element_type=jnp.float32)


def _tc_last_body(d_out, p_ref, g_ref, dinv_ref, b_ref, out_ref):
    z = p_ref[0, :, :d_out] + p_ref[1, :, :d_out] + g_ref[:, :d_out]
    z = jnp.maximum(dinv_ref[...] * z + b_ref[...], 0.0)
    m = jnp.max(z, axis=1, keepdims=True)
    lse = m + jnp.log(jnp.sum(jnp.exp(z - m), axis=1, keepdims=True))
    out_ref[...] = z - lse


def _tc_mm(x, w):
    n, d_in = x.shape
    d_out = w.shape[1]
    return pl.pallas_call(
        _tc_mm_body,
        grid=(n // BLK,),
        in_specs=[
            pl.BlockSpec((BLK, d_in), lambda i: (i, 0)),
            pl.BlockSpec((d_in, d_out), lambda i: (0, 0)),
        ],
        out_specs=pl.BlockSpec((BLK, d_out), lambda i: (i, 0)),
        out_shape=jax.ShapeDtypeStruct((n, d_out), jnp.float32),
    )(x, w)


def _tc_scale(dp, h):
    n, d = h.shape
    return pl.pallas_call(
        _tc_scale_body,
        grid=(n // BLK,),
        in_specs=[
            pl.BlockSpec((NC, BLK, dp.shape[2]), lambda i: (0, i, 0)),
            pl.BlockSpec((BLK, d), lambda i: (i, 0)),
        ],
        out_specs=[
            pl.BlockSpec((BLK, d), lambda i: (i, 0)),
            pl.BlockSpec((BLK, 1), lambda i: (i, 0)),
        ],
        out_shape=[
            jax.ShapeDtypeStruct((n, d), jnp.float32),
            jax.ShapeDtypeStruct((n, 1), jnp.float32),
        ],
    )(dp, h)


def _tc_mid(p, g, dinv, b, w):
    n, d = g.shape
    d_out = w.shape[1]
    return pl.pallas_call(
        _tc_mid_body,
        grid=(n // BLK,),
        in_specs=[
            pl.BlockSpec((NC, BLK, d), lambda i: (0, i, 0)),
            pl.BlockSpec((BLK, d), lambda i: (i, 0)),
            pl.BlockSpec((BLK, 1), lambda i: (i, 0)),
            pl.BlockSpec((1, d), lambda i: (0, 0)),
            pl.BlockSpec((d, d_out), lambda i: (0, 0)),
        ],
        out_specs=pl.BlockSpec((BLK, d_out), lambda i: (i, 0)),
        out_shape=jax.ShapeDtypeStruct((n, d_out), jnp.float32),
    )(p, g, dinv, b, w)


def _tc_last(p, g, dinv, b, d_out):
    n, d = g.shape
    return pl.pallas_call(
        functools.partial(_tc_last_body, d_out),
        grid=(n // BLK,),
        in_specs=[
            pl.BlockSpec((NC, BLK, d), lambda i: (0, i, 0)),
            pl.BlockSpec((BLK, d), lambda i: (i, 0)),
            pl.BlockSpec((BLK, 1), lambda i: (i, 0)),
            pl.BlockSpec((1, d_out), lambda i: (0, 0)),
        ],
        out_specs=pl.BlockSpec((BLK, d_out), lambda i: (i, 0)),
        out_shape=jax.ShapeDtypeStruct((n, d_out), jnp.float32),
    )(p, g, dinv, b)


def kernel(x, edge_index, W1, b1, W2, b2, W3, b3):
    n, d_in = x.shape
    e = edge_index.shape[1]
    d_hid = W2.shape[0]
    d_out = W3.shape[1]

    e_pad, _ = _padded_chunks(e)
    pad_src = jnp.arange(e_pad - e, dtype=jnp.int32) % n
    src2 = jnp.concatenate([edge_index[0], pad_src]).reshape(-1, CH)
    pad_dst = n + (jnp.arange(e_pad - e, dtype=jnp.int32) % 128)
    dst2 = jnp.concatenate([edge_index[1], pad_dst]).reshape(-1, CH)

    zeros_wide = jnp.zeros((ROWS_MAIN, max(d_in, d_hid)), jnp.float32)
    ones128 = jnp.ones((CH, 128), jnp.float32)

    sc_degree = _make_sc_degree(n, e)
    sc_spmm_h = _make_sc_spmm(n, e, d_hid)

    W3p = jnp.pad(W3, ((0, 0), (0, d_hid - d_out)))

    dp = sc_degree(dst2, ones128, zeros_wide)
    h1 = _tc_mm(x, W1)
    g1, dinv = _tc_scale(dp, h1)
    p1 = sc_spmm_h(src2, dst2, g1, zeros_wide)
    g2 = _tc_mid(p1, g1, dinv, b1.reshape(1, -1), W2)
    p2 = sc_spmm_h(src2, dst2, g2, zeros_wide)
    g3 = _tc_mid(p2, g2, dinv, b2.reshape(1, -1), W3p)
    p3 = sc_spmm_h(src2, dst2, g3, zeros_wide)
    return _tc_last(p3, g3, dinv, b3.reshape(1, -1), d_out)
